# baseline (speedup 1.0000x reference)
import functools

import jax
import jax.numpy as jnp
from jax.experimental import pallas as pl
from jax.experimental.pallas import tpu as pltpu

_BF = jnp.bfloat16
_F32 = jnp.float32


def _split(a):
    hi = a.astype(_BF)
    lo = (a - hi.astype(_F32)).astype(_BF)
    return hi, lo


def _dot(a, b):
    return jax.lax.dot_general(a, b, (((1,), (0,)), ((), ())),
                               preferred_element_type=_F32)


def _dot3(a, bh, bl):
    ah, al = _split(a)
    return _dot(ah, bh) + _dot(ah, bl) + _dot(al, bh)


def _body(x_ref, w1h_hbm, w1l_hbm, w2h_hbm, w2l_hbm, e2h_hbm, e2l_hbm,
          e2th_hbm, e2tl_hbm, w3_hbm, w4_hbm, bias_hbm,
          recon_ref, ze_ref, embout_ref,
          w1h_v, w1l_v, w2h_v, w2l_v, e2h_v, e2l_v,
          e2th_v, e2tl_v, w3_v, w4_v, bias_v, sem, *, K, P, F):
    @pl.when(pl.program_id(0) == 0)
    def _load_weights():
        for src, dst in ((w1h_hbm, w1h_v), (w1l_hbm, w1l_v),
                         (w2h_hbm, w2h_v), (w2l_hbm, w2l_v),
                         (e2h_hbm, e2h_v), (e2l_hbm, e2l_v),
                         (e2th_hbm, e2th_v), (e2tl_hbm, e2tl_v),
                         (w3_hbm, w3_v), (w4_hbm, w4_v),
                         (bias_hbm, bias_v)):
            cp = pltpu.make_async_copy(src, dst, sem)
            cp.start()
            cp.wait()

    H = P * K
    b1 = bias_v[0:1, :F]
    b2 = bias_v[1:2, :H]
    b3 = bias_v[2:3, :F]
    b4 = bias_v[3:4, :]
    e2c = bias_v[4:5, :H]

    h1 = jnp.maximum(_dot3(x_ref[...], w1h_v[...], w1l_v[...]) + b1, 0.0)
    ze = _dot3(h1, w2h_v[...], w2l_v[...]) + b2
    ze_ref[...] = ze

    cross = _dot3(ze, e2h_v[...], e2l_v[...])
    scores = e2c - 2.0 * cross

    iota = jax.lax.broadcasted_iota(jnp.int32, (scores.shape[0], K), 1)
    ohs = []
    for p in range(P):
        s = scores[:, p * K:(p + 1) * K]
        m = jnp.min(s, axis=1, keepdims=True)
        cand = jnp.where(s == m, iota, K)
        kmin = jnp.min(cand, axis=1, keepdims=True)
        ohs.append((iota == kmin).astype(_BF))
    oh = jnp.concatenate(ohs, axis=1)
    zq = _dot(oh, e2th_v[...]) + _dot(oh, e2tl_v[...])
    embout_ref[...] = zq

    h3 = jnp.maximum(_dot(zq.astype(_BF), w3_v[...]) + b3, 0.0)
    logits = _dot(h3.astype(_BF), w4_v[...]) + b4
    recon_ref[...] = jax.nn.sigmoid(logits)


def kernel(x, W1, b1, W2, b2, W3, b3, W4, b4, emb_weight):
    B, L = x.shape
    D, K = emb_weight.shape
    H = W2.shape[0]
    P = H // D
    F1 = W1.shape[0]
    F = 512
    BB = 512

    def padto(a, rows, cols):
        return jnp.pad(a, ((0, rows - a.shape[0]), (0, cols - a.shape[1])))

    W1p = padto(W1.T, L, F)
    W2p = padto(W2.T, F, H)
    W3p = padto(W3.T, H, F)
    W4p = padto(W4.T, F, L)

    eye = jnp.eye(P, dtype=_F32)
    E2 = (emb_weight[:, None, None, :] * eye[None, :, :, None]
          ).reshape(D * P, P * K)
    E2T = (eye[:, None, None, :] * emb_weight.T[None, :, :, None]
           ).reshape(P * K, D * P)
    e2c = jnp.sum(emb_weight * emb_weight, axis=0)
    e2c = jnp.tile(e2c, P)

    W1h, W1l = _split(W1p)
    W2h, W2l = _split(W2p)
    E2h, E2l = _split(E2)
    E2Th, E2Tl = _split(E2T)
    W3b = W3p.astype(_BF)
    W4b = W4p.astype(_BF)

    def padrow(v):
        return jnp.pad(v, (0, L - v.shape[0]))

    bias = jnp.stack([padrow(b1), padrow(b2), padrow(b3), b4,
                      padrow(e2c), jnp.zeros((L,), _F32),
                      jnp.zeros((L,), _F32), jnp.zeros((L,), _F32)])

    grid = (B // BB,)
    row = lambda shape: pl.BlockSpec(shape, lambda i: (i, 0))
    anyspec = pl.BlockSpec(memory_space=pltpu.MemorySpace.HBM)

    recon, ze, embout = pl.pallas_call(
        functools.partial(_body, K=K, P=P, F=F),
        grid=grid,
        in_specs=[row((BB, L))] + [anyspec] * 11,
        out_specs=(row((BB, L)), row((BB, H)), row((BB, H))),
        out_shape=(
            jax.ShapeDtypeStruct((B, L), x.dtype),
            jax.ShapeDtypeStruct((B, H), x.dtype),
            jax.ShapeDtypeStruct((B, H), x.dtype),
        ),
        scratch_shapes=[
            pltpu.VMEM((L, F), _BF), pltpu.VMEM((L, F), _BF),
            pltpu.VMEM((F, H), _BF), pltpu.VMEM((F, H), _BF),
            pltpu.VMEM((H, P * K), _BF), pltpu.VMEM((H, P * K), _BF),
            pltpu.VMEM((P * K, H), _BF), pltpu.VMEM((P * K, H), _BF),
            pltpu.VMEM((H, F), _BF), pltpu.VMEM((F, L), _BF),
            pltpu.VMEM((8, L), _F32),
            pltpu.SemaphoreType.DMA,
        ],
        compiler_params=pltpu.CompilerParams(
            dimension_semantics=("arbitrary",)),
    )(x, W1h, W1l, W2h, W2l, E2h, E2l, E2Th, E2Tl, W3b, W4b, bias)

    return recon, ze.reshape(B, D, P), embout


# speedup vs baseline: 1.4234x; 1.4234x over previous
import functools

import jax
import jax.numpy as jnp
from jax.experimental import pallas as pl
from jax.experimental.pallas import tpu as pltpu

_BF = jnp.bfloat16
_F32 = jnp.float32


def _trunc16(a):
    ai = jax.lax.bitcast_convert_type(a, jnp.int32)
    return jax.lax.bitcast_convert_type(ai & jnp.int32(-65536), _F32)


def _split3(a):
    af = _trunc16(a)
    r1 = a - af
    bf = _trunc16(r1)
    r2 = r1 - bf
    return af.astype(_BF), bf.astype(_BF), r2.astype(_BF)


def _dot(a, b):
    return jax.lax.dot_general(a, b, (((1,), (0,)), ((), ())),
                               preferred_element_type=_F32)


def _bdot(a, b_ref):
    return _dot(a.astype(_BF), b_ref[...])


def _body(x_ref, w1_ref, w2_ref, e2_ref, e2ta_ref, e2tb_ref, e2tc_ref,
          w3_ref, w4_ref,
          b1_ref, b2_ref, b3_ref, b4_ref, e2c_ref,
          recon_ref, ze_ref, embout_ref, *, K, P, F):
    h1 = jnp.maximum(_bdot(x_ref[...], w1_ref) + b1_ref[...], 0.0)
    ze = _bdot(h1, w2_ref) + b2_ref[...]
    ze_ref[...] = ze

    scores = e2c_ref[...] - 2.0 * _bdot(ze, e2_ref)

    iota = jax.lax.broadcasted_iota(jnp.int32, (scores.shape[0], K), 1)
    ohs = []
    for p in range(P):
        s = scores[:, p * K:(p + 1) * K]
        m = jnp.min(s, axis=1, keepdims=True)
        cand = jnp.where(s == m, iota, K)
        kmin = jnp.min(cand, axis=1, keepdims=True)
        ohs.append((iota == kmin).astype(_BF))
    oh = jnp.concatenate(ohs, axis=1)
    zq = ((_dot(oh, e2ta_ref[...]) + _dot(oh, e2tb_ref[...]))
          + _dot(oh, e2tc_ref[...]))
    embout_ref[...] = zq

    h3 = jnp.maximum(_bdot(zq, w3_ref) + b3_ref[...], 0.0)
    recon_ref[...] = jax.nn.sigmoid(_bdot(h3, w4_ref) + b4_ref[...])


def kernel(x, W1, b1, W2, b2, W3, b3, W4, b4, emb_weight):
    B, L = x.shape
    D, K = emb_weight.shape
    H = W2.shape[0]
    P = H // D
    F1 = W1.shape[0]
    F = 512
    BB = 512

    def padto(a, rows, cols):
        return jnp.pad(a, ((0, rows - a.shape[0]), (0, cols - a.shape[1])))

    W1b = padto(W1.T, L, F).astype(_BF)
    W2b = padto(W2.T, F, H).astype(_BF)
    W3b = padto(W3.T, H, F).astype(_BF)
    W4b = padto(W4.T, F, L).astype(_BF)

    eye = jnp.eye(P, dtype=_F32)
    E2 = (emb_weight[:, None, None, :] * eye[None, :, :, None]
          ).reshape(D * P, P * K)
    E2T = (eye[:, None, None, :] * emb_weight.T[None, :, :, None]
           ).reshape(P * K, D * P)
    E2b = E2.astype(_BF)
    E2Ta, E2Tb, E2Tc = _split3(E2T)
    e2c = jnp.sum(emb_weight * emb_weight, axis=0)
    e2c = jnp.tile(e2c, P)

    b1r = jnp.pad(b1, (0, F - F1)).reshape(1, F)
    b2r = b2.reshape(1, H)
    b3r = jnp.pad(b3, (0, F - F1)).reshape(1, F)
    b4r = b4.reshape(1, L)
    e2cr = e2c.reshape(1, P * K)

    grid = (B // BB,)
    full = lambda shape: pl.BlockSpec(shape, lambda i: (0, 0))
    row = lambda shape: pl.BlockSpec(shape, lambda i: (i, 0))

    recon, ze, embout = pl.pallas_call(
        functools.partial(_body, K=K, P=P, F=F),
        grid=grid,
        in_specs=[
            row((BB, L)),
            full((L, F)),
            full((F, H)),
            full((H, P * K)),
            full((P * K, H)), full((P * K, H)), full((P * K, H)),
            full((H, F)), full((F, L)),
            full((1, F)), full((1, H)), full((1, F)), full((1, L)),
            full((1, P * K)),
        ],
        out_specs=(row((BB, L)), row((BB, H)), row((BB, H))),
        out_shape=(
            jax.ShapeDtypeStruct((B, L), x.dtype),
            jax.ShapeDtypeStruct((B, H), x.dtype),
            jax.ShapeDtypeStruct((B, H), x.dtype),
        ),
        compiler_params=pltpu.CompilerParams(
            dimension_semantics=("arbitrary",)),
    )(x, W1b, W2b, E2b, E2Ta, E2Tb, E2Tc, W3b, W4b,
      b1r, b2r, b3r, b4r, e2cr)

    return recon, ze.reshape(B, D, P), embout
